# Initial kernel scaffold; baseline (speedup 1.0000x reference)
import jax
import jax.numpy as jnp
from jax.experimental import pallas as pl


def kernel(ent_emb, rel_emb, vis_feats, txt_feats, W_vis, W_txt, head, relation, tail):
    raise NotImplementedError("write your pallas kernel here")


# speedup vs baseline: 7.6469x; 7.6469x over previous
import functools

import jax
import jax.numpy as jnp
from jax import lax
from jax.experimental import pallas as pl
from jax.experimental.pallas import tpu as pltpu
from jax.experimental.pallas import tpu_sc as plsc

_DIM = 128
_MARGIN = 6.0
_EPSILON = 2.0
_PI = 3.141592653589793

_NC = 2
_NS = 16
_NW = _NC * _NS


@functools.lru_cache(maxsize=None)
def _sc_gather_fn(B2, Brel, D2, F, DR):
    rows_w = B2 // _NW
    C = 128
    n_ch = rows_w // C
    rel_w = Brel // _NW
    n_rel_ch = rel_w // C

    mesh = plsc.VectorSubcoreMesh(core_axis_name="c", subcore_axis_name="s")

    @functools.partial(
        pl.kernel,
        mesh=mesh,
        out_type=[
            jax.ShapeDtypeStruct((B2, D2 + 2 * F), jnp.float32),
            jax.ShapeDtypeStruct((Brel, DR), jnp.float32),
        ],
        scratch_types=[
            pltpu.VMEM((C,), jnp.int32),
            pltpu.VMEM((C, D2), jnp.float32),
            pltpu.VMEM((C, F), jnp.float32),
            pltpu.VMEM((C, F), jnp.float32),
            pltpu.VMEM((C,), jnp.int32),
            pltpu.VMEM((C, DR), jnp.float32),
            pltpu.SemaphoreType.DMA,
            pltpu.SemaphoreType.DMA,
            pltpu.SemaphoreType.DMA,
        ],
    )
    def gather_kernel(ent_hbm, vis_hbm, txt_hbm, rel_hbm, idx_hbm, ridx_hbm,
                      cat_hbm, rg_hbm,
                      idx_v, ent_b, vis_b, txt_b, ridx_v, rel_b,
                      sem0, sem1, sem2):
        wid = lax.axis_index("s") * _NC + lax.axis_index("c")
        base = pl.multiple_of(wid * rows_w, 8)
        for ci in range(n_ch):
            r0 = pl.multiple_of(base + ci * C, 8)
            pltpu.sync_copy(idx_hbm.at[pl.ds(r0, C)], idx_v)
            cp0 = pltpu.async_copy(ent_hbm.at[idx_v], ent_b, sem0)
            cp1 = pltpu.async_copy(vis_hbm.at[idx_v], vis_b, sem1)
            cp2 = pltpu.async_copy(txt_hbm.at[idx_v], txt_b, sem2)
            cp0.wait()
            cp1.wait()
            cp2.wait()
            pltpu.sync_copy(ent_b, cat_hbm.at[pl.ds(r0, C), pl.ds(0, D2)])
            pltpu.sync_copy(vis_b, cat_hbm.at[pl.ds(r0, C), pl.ds(D2, F)])
            pltpu.sync_copy(txt_b, cat_hbm.at[pl.ds(r0, C), pl.ds(D2 + F, F)])
        rbase = pl.multiple_of(wid * rel_w, 8)
        for ci in range(n_rel_ch):
            r0 = pl.multiple_of(rbase + ci * C, 8)
            pltpu.sync_copy(ridx_hbm.at[pl.ds(r0, C)], ridx_v)
            pltpu.async_copy(rel_hbm.at[ridx_v], rel_b, sem0).wait()
            pltpu.sync_copy(rel_b, rg_hbm.at[pl.ds(r0, C)])

    return gather_kernel


@functools.lru_cache(maxsize=None)
def _tc_score_fn(B, D2, F, DR):
    BLK = 512
    nblk = B // BLK
    CAT = D2 + 2 * F
    inv3 = 1.0 / 3.0
    phase_scale = _PI * _DIM / (_MARGIN + _EPSILON)

    def body(h_ref, t_ref, r_ref, w_ref, o_ref):
        w = w_ref[...]
        h = h_ref[...]
        t = t_ref[...]
        mh = (jnp.dot(h[:, D2:], w, preferred_element_type=jnp.float32)
              + h[:, :D2]) * inv3
        mt = (jnp.dot(t[:, D2:], w, preferred_element_type=jnp.float32)
              + t[:, :D2]) * inv3
        phase = r_ref[...] * phase_scale
        re_r = jnp.cos(phase)
        im_r = jnp.sin(phase)
        re_h, im_h = mh[:, :_DIM], mh[:, _DIM:]
        re_t, im_t = mt[:, :_DIM], mt[:, _DIM:]
        re_s = re_h * re_r - im_h * im_r - re_t
        im_s = re_h * im_r + im_h * re_r - im_t
        dist = jnp.sum(jnp.sqrt(re_s * re_s + im_s * im_s + 1e-12), axis=1)
        o_ref[...] = _MARGIN - dist

    return pl.pallas_call(
        body,
        grid=(nblk,),
        in_specs=[
            pl.BlockSpec((BLK, CAT), lambda i: (i, 0)),
            pl.BlockSpec((BLK, CAT), lambda i: (i + nblk, 0)),
            pl.BlockSpec((BLK, DR), lambda i: (i, 0)),
            pl.BlockSpec((2 * F, D2), lambda i: (0, 0)),
        ],
        out_specs=pl.BlockSpec((BLK,), lambda i: (i,)),
        out_shape=jax.ShapeDtypeStruct((B,), jnp.float32),
    )


def kernel(ent_emb, rel_emb, vis_feats, txt_feats, W_vis, W_txt,
           head, relation, tail):
    B = head.shape[0]
    D2 = ent_emb.shape[1]
    F = vis_feats.shape[1]
    DR = rel_emb.shape[1]
    idx_all = jnp.concatenate([head, tail]).astype(jnp.int32)
    rel_idx = relation.astype(jnp.int32)
    W_vt = jnp.concatenate([W_vis, W_txt], axis=0)
    cat, rg = _sc_gather_fn(2 * B, B, D2, F, DR)(
        ent_emb, vis_feats, txt_feats, rel_emb, idx_all, rel_idx)
    return _tc_score_fn(B, D2, F, DR)(cat, cat, rg, W_vt)


# speedup vs baseline: 7.8749x; 1.0298x over previous
import functools

import jax
import jax.numpy as jnp
from jax import lax
from jax.experimental import pallas as pl
from jax.experimental.pallas import tpu as pltpu
from jax.experimental.pallas import tpu_sc as plsc

_DIM = 128
_MARGIN = 6.0
_EPSILON = 2.0
_PI = 3.141592653589793

_NC = 2
_NS = 16
_NW = _NC * _NS


def _pipelined_gather(tables, dsts, idx_full, bufs, gsems, wsems,
                      base, rows_w, C):
    n_ch = rows_w // C
    gath = {}
    writes = {}

    def start_gather(ci):
        slot = ci % 2
        idxs = idx_full.at[pl.ds(ci * C, C)]
        gath[ci] = [pltpu.async_copy(tbl.at[idxs], buf, gsems[slot])
                    for tbl, buf in zip(tables, bufs[slot])]

    start_gather(0)
    for ci in range(n_ch):
        slot = ci % 2
        for c in gath.pop(ci):
            c.wait()
        r0 = base + ci * C
        writes[ci] = [pltpu.async_copy(buf, dst.at[pl.ds(r0, C)], wsems[slot])
                      for dst, buf in zip(dsts, bufs[slot])]
        if ci + 1 < n_ch:
            if ci - 1 >= 0:
                for c in writes.pop(ci - 1):
                    c.wait()
            start_gather(ci + 1)
    for ci in sorted(writes):
        for c in writes[ci]:
            c.wait()


@functools.lru_cache(maxsize=None)
def _sc_gather_fn(B2, Brel, D2, F, DR):
    rows_w = B2 // _NW
    C = 64
    rel_w = Brel // _NW

    mesh = plsc.VectorSubcoreMesh(core_axis_name="c", subcore_axis_name="s")

    @functools.partial(
        pl.kernel,
        mesh=mesh,
        out_type=[
            jax.ShapeDtypeStruct((B2, D2), jnp.float32),
            jax.ShapeDtypeStruct((B2, F), jnp.float32),
            jax.ShapeDtypeStruct((B2, F), jnp.float32),
            jax.ShapeDtypeStruct((Brel, DR), jnp.float32),
        ],
        scratch_types=[
            pltpu.VMEM((rows_w,), jnp.int32),
            pltpu.VMEM((rel_w,), jnp.int32),
            pltpu.VMEM((C, D2), jnp.float32),
            pltpu.VMEM((C, F), jnp.float32),
            pltpu.VMEM((C, F), jnp.float32),
            pltpu.VMEM((C, D2), jnp.float32),
            pltpu.VMEM((C, F), jnp.float32),
            pltpu.VMEM((C, F), jnp.float32),
            pltpu.VMEM((C, DR), jnp.float32),
            pltpu.VMEM((C, DR), jnp.float32),
            pltpu.SemaphoreType.DMA,
            pltpu.SemaphoreType.DMA,
            pltpu.SemaphoreType.DMA,
            pltpu.SemaphoreType.DMA,
        ],
    )
    def gather_kernel(ent_hbm, vis_hbm, txt_hbm, rel_hbm, idx_hbm, ridx_hbm,
                      ge_hbm, gv_hbm, gt_hbm, rg_hbm,
                      idx_full, ridx_full,
                      ent_b0, vis_b0, txt_b0, ent_b1, vis_b1, txt_b1,
                      rel_b0, rel_b1,
                      gsem0, gsem1, wsem0, wsem1):
        wid = lax.axis_index("s") * _NC + lax.axis_index("c")
        base = pl.multiple_of(wid * rows_w, 8)
        pltpu.sync_copy(idx_hbm.at[pl.ds(base, rows_w)], idx_full)
        _pipelined_gather(
            [ent_hbm, vis_hbm, txt_hbm], [ge_hbm, gv_hbm, gt_hbm],
            idx_full,
            [[ent_b0, vis_b0, txt_b0], [ent_b1, vis_b1, txt_b1]],
            [gsem0, gsem1], [wsem0, wsem1], base, rows_w, C)
        rbase = pl.multiple_of(wid * rel_w, 8)
        pltpu.sync_copy(ridx_hbm.at[pl.ds(rbase, rel_w)], ridx_full)
        _pipelined_gather(
            [rel_hbm], [rg_hbm], ridx_full,
            [[rel_b0], [rel_b1]],
            [gsem0, gsem1], [wsem0, wsem1], rbase, rel_w, C)

    return gather_kernel


@functools.lru_cache(maxsize=None)
def _tc_score_fn(B, D2, F, DR):
    BLK = 512
    nblk = B // BLK
    inv3 = 1.0 / 3.0
    phase_scale = _PI * _DIM / (_MARGIN + _EPSILON)

    def body(he_ref, hv_ref, ht_ref, te_ref, tv_ref, tt_ref, r_ref,
             wv_ref, wt_ref, o_ref):
        wv = wv_ref[...]
        wt = wt_ref[...]
        mh = (jnp.dot(hv_ref[...], wv, preferred_element_type=jnp.float32)
              + jnp.dot(ht_ref[...], wt, preferred_element_type=jnp.float32)
              + he_ref[...]) * inv3
        mt = (jnp.dot(tv_ref[...], wv, preferred_element_type=jnp.float32)
              + jnp.dot(tt_ref[...], wt, preferred_element_type=jnp.float32)
              + te_ref[...]) * inv3
        phase = r_ref[...] * phase_scale
        re_r = jnp.cos(phase)
        im_r = jnp.sin(phase)
        re_h, im_h = mh[:, :_DIM], mh[:, _DIM:]
        re_t, im_t = mt[:, :_DIM], mt[:, _DIM:]
        re_s = re_h * re_r - im_h * im_r - re_t
        im_s = re_h * im_r + im_h * re_r - im_t
        dist = jnp.sum(jnp.sqrt(re_s * re_s + im_s * im_s + 1e-12), axis=1)
        o_ref[...] = _MARGIN - dist

    return pl.pallas_call(
        body,
        grid=(nblk,),
        in_specs=[
            pl.BlockSpec((BLK, D2), lambda i: (i, 0)),
            pl.BlockSpec((BLK, F), lambda i: (i, 0)),
            pl.BlockSpec((BLK, F), lambda i: (i, 0)),
            pl.BlockSpec((BLK, D2), lambda i: (i + nblk, 0)),
            pl.BlockSpec((BLK, F), lambda i: (i + nblk, 0)),
            pl.BlockSpec((BLK, F), lambda i: (i + nblk, 0)),
            pl.BlockSpec((BLK, DR), lambda i: (i, 0)),
            pl.BlockSpec((F, D2), lambda i: (0, 0)),
            pl.BlockSpec((F, D2), lambda i: (0, 0)),
        ],
        out_specs=pl.BlockSpec((BLK,), lambda i: (i,)),
        out_shape=jax.ShapeDtypeStruct((B,), jnp.float32),
    )


def kernel(ent_emb, rel_emb, vis_feats, txt_feats, W_vis, W_txt,
           head, relation, tail):
    B = head.shape[0]
    D2 = ent_emb.shape[1]
    F = vis_feats.shape[1]
    DR = rel_emb.shape[1]
    idx_all = jnp.concatenate([head, tail]).astype(jnp.int32)
    rel_idx = relation.astype(jnp.int32)
    ge, gv, gt, rg = _sc_gather_fn(2 * B, B, D2, F, DR)(
        ent_emb, vis_feats, txt_feats, rel_emb, idx_all, rel_idx)
    return _tc_score_fn(B, D2, F, DR)(ge, gv, gt, ge, gv, gt, rg,
                                      W_vis, W_txt)


# speedup vs baseline: 8.0945x; 1.0279x over previous
import functools

import jax
import jax.numpy as jnp
from jax import lax
from jax.experimental import pallas as pl
from jax.experimental.pallas import tpu as pltpu
from jax.experimental.pallas import tpu_sc as plsc

_DIM = 128
_MARGIN = 6.0
_EPSILON = 2.0
_PI = 3.141592653589793

_NC = 2
_NS = 16
_NW = _NC * _NS


def _pipelined_gather(tables, dsts, idx_full, bufs, gsems, wsems,
                      base, rows_w, C):
    n_ch = rows_w // C
    gath = {}
    writes = {}

    def start_gather(ci):
        slot = ci % 2
        idxs = idx_full.at[pl.ds(ci * C, C)]
        gath[ci] = [pltpu.async_copy(tbl.at[idxs], buf, gsems[slot])
                    for tbl, buf in zip(tables, bufs[slot])]

    start_gather(0)
    for ci in range(n_ch):
        slot = ci % 2
        for c in gath.pop(ci):
            c.wait()
        r0 = base + ci * C
        writes[ci] = [pltpu.async_copy(buf, dst.at[pl.ds(r0, C)], wsems[slot])
                      for dst, buf in zip(dsts, bufs[slot])]
        if ci + 1 < n_ch:
            if ci - 1 >= 0:
                for c in writes.pop(ci - 1):
                    c.wait()
            start_gather(ci + 1)
    for ci in sorted(writes):
        for c in writes[ci]:
            c.wait()


@functools.lru_cache(maxsize=None)
def _sc_gather_fn(B2, Brel, D2, F, DR):
    rows_w = B2 // _NW
    C = 64
    rel_w = Brel // _NW

    mesh = plsc.VectorSubcoreMesh(core_axis_name="c", subcore_axis_name="s")

    @functools.partial(
        pl.kernel,
        mesh=mesh,
        out_type=[
            jax.ShapeDtypeStruct((B2, D2), jnp.float32),
            jax.ShapeDtypeStruct((B2, F), jnp.float32),
            jax.ShapeDtypeStruct((B2, F), jnp.float32),
            jax.ShapeDtypeStruct((Brel, DR), jnp.float32),
        ],
        scratch_types=[
            pltpu.VMEM((rows_w,), jnp.int32),
            pltpu.VMEM((rel_w,), jnp.int32),
            pltpu.VMEM((C, D2), jnp.float32),
            pltpu.VMEM((C, F), jnp.float32),
            pltpu.VMEM((C, F), jnp.float32),
            pltpu.VMEM((C, D2), jnp.float32),
            pltpu.VMEM((C, F), jnp.float32),
            pltpu.VMEM((C, F), jnp.float32),
            pltpu.VMEM((C, DR), jnp.float32),
            pltpu.VMEM((C, DR), jnp.float32),
            pltpu.SemaphoreType.DMA,
            pltpu.SemaphoreType.DMA,
            pltpu.SemaphoreType.DMA,
            pltpu.SemaphoreType.DMA,
        ],
    )
    def gather_kernel(ent_hbm, vis_hbm, txt_hbm, rel_hbm, idx_hbm, ridx_hbm,
                      ge_hbm, gv_hbm, gt_hbm, rg_hbm,
                      idx_full, ridx_full,
                      ent_b0, vis_b0, txt_b0, ent_b1, vis_b1, txt_b1,
                      rel_b0, rel_b1,
                      gsem0, gsem1, wsem0, wsem1):
        wid = lax.axis_index("s") * _NC + lax.axis_index("c")
        base = pl.multiple_of(wid * rows_w, 8)
        pltpu.sync_copy(idx_hbm.at[pl.ds(base, rows_w)], idx_full)
        _pipelined_gather(
            [ent_hbm, vis_hbm, txt_hbm], [ge_hbm, gv_hbm, gt_hbm],
            idx_full,
            [[ent_b0, vis_b0, txt_b0], [ent_b1, vis_b1, txt_b1]],
            [gsem0, gsem1], [wsem0, wsem1], base, rows_w, C)
        rbase = pl.multiple_of(wid * rel_w, 8)
        pltpu.sync_copy(ridx_hbm.at[pl.ds(rbase, rel_w)], ridx_full)
        _pipelined_gather(
            [rel_hbm], [rg_hbm], ridx_full,
            [[rel_b0], [rel_b1]],
            [gsem0, gsem1], [wsem0, wsem1], rbase, rel_w, C)

    return gather_kernel


@functools.lru_cache(maxsize=None)
def _tc_score_fn(B, D2, F, DR):
    BLK = 512
    nblk = B // BLK
    inv3 = 1.0 / 3.0
    phase_scale = _PI * _DIM / (_MARGIN + _EPSILON)

    def body(he_ref, hv_ref, ht_ref, te_ref, tv_ref, tt_ref, r_ref,
             wv_ref, wt_ref, o_ref):
        wv = wv_ref[...]
        wt = wt_ref[...]
        mh = (jnp.dot(hv_ref[...], wv, preferred_element_type=jnp.float32)
              + jnp.dot(ht_ref[...], wt, preferred_element_type=jnp.float32)
              + he_ref[...]) * inv3
        mt = (jnp.dot(tv_ref[...], wv, preferred_element_type=jnp.float32)
              + jnp.dot(tt_ref[...], wt, preferred_element_type=jnp.float32)
              + te_ref[...]) * inv3
        phase = r_ref[...] * phase_scale
        re_r = jnp.cos(phase)
        im_r = jnp.sin(phase)
        re_h, im_h = mh[:, :_DIM], mh[:, _DIM:]
        re_t, im_t = mt[:, :_DIM], mt[:, _DIM:]
        re_s = re_h * re_r - im_h * im_r - re_t
        im_s = re_h * im_r + im_h * re_r - im_t
        dist = jnp.sum(jnp.sqrt(re_s * re_s + im_s * im_s + 1e-12), axis=1)
        o_ref[...] = _MARGIN - dist

    return pl.pallas_call(
        body,
        grid=(nblk,),
        in_specs=[
            pl.BlockSpec((BLK, D2), lambda i: (i, 0)),
            pl.BlockSpec((BLK, F), lambda i: (i, 0)),
            pl.BlockSpec((BLK, F), lambda i: (i, 0)),
            pl.BlockSpec((BLK, D2), lambda i: (i + nblk, 0)),
            pl.BlockSpec((BLK, F), lambda i: (i + nblk, 0)),
            pl.BlockSpec((BLK, F), lambda i: (i + nblk, 0)),
            pl.BlockSpec((BLK, DR), lambda i: (i, 0)),
            pl.BlockSpec((F, D2), lambda i: (0, 0)),
            pl.BlockSpec((F, D2), lambda i: (0, 0)),
        ],
        out_specs=pl.BlockSpec((BLK,), lambda i: (i,)),
        out_shape=jax.ShapeDtypeStruct((B,), jnp.float32),
    )


_NSPLIT = 4


def kernel(ent_emb, rel_emb, vis_feats, txt_feats, W_vis, W_txt,
           head, relation, tail):
    B = head.shape[0]
    D2 = ent_emb.shape[1]
    F = vis_feats.shape[1]
    DR = rel_emb.shape[1]
    head = head.astype(jnp.int32)
    tail = tail.astype(jnp.int32)
    rel_idx = relation.astype(jnp.int32)
    Bp = B // _NSPLIT
    sc = _sc_gather_fn(2 * Bp, Bp, D2, F, DR)
    tc = _tc_score_fn(Bp, D2, F, DR)
    outs = []
    for p in range(_NSPLIT):
        sl = slice(p * Bp, (p + 1) * Bp)
        idx_p = jnp.concatenate([head[sl], tail[sl]])
        ge, gv, gt, rg = sc(ent_emb, vis_feats, txt_feats, rel_emb,
                            idx_p, rel_idx[sl])
        outs.append(tc(ge, gv, gt, ge, gv, gt, rg, W_vis, W_txt))
    return jnp.concatenate(outs)
